# baseline (speedup 1.0000x reference)
import functools

import jax
import jax.numpy as jnp
from jax import lax
from jax.experimental import pallas as pl
from jax.experimental.pallas import tpu as pltpu
from jax.experimental.pallas import tpu_sc as plsc

N = 100000
L = 200000
NW = 32
LANES = 16

LINK_CHUNK = 6272
WORDS = L // 2
NODE_CHUNK = 3136
ARR_SPLIT = 4
ARR_CHUNK = N // ARR_SPLIT
W_CHUNK = WORDS // ARR_SPLIT

_mesh = plsc.VectorSubcoreMesh(core_axis_name="c", subcore_axis_name="s")


def _wid():
    return lax.axis_index("s") * 2 + lax.axis_index("c")


def _bf16_hi(g):
    b = plsc.bitcast(g, jnp.int32)
    return (b + 0x8000) & jnp.int32(-65536)


def _grad_body(head_hbm, tail_hbm, len_hbm, array_hbm, w_hbm,
               arr_v, idxh_v, idxt_v, len_v, w_v, sem):
    wid = _wid()
    base = pl.multiple_of(
        jnp.where(wid == NW - 1, L - LINK_CHUNK, wid * LINK_CHUNK), 64)
    copies = [pltpu.async_copy(array_hbm.at[pl.ds(c * ARR_CHUNK, ARR_CHUNK)],
                               arr_v.at[pl.ds(c * ARR_CHUNK, ARR_CHUNK)], sem)
              for c in range(ARR_SPLIT)]
    copies.append(pltpu.async_copy(
        head_hbm.at[pl.ds(base, LINK_CHUNK)], idxh_v, sem))
    copies.append(pltpu.async_copy(
        tail_hbm.at[pl.ds(base, LINK_CHUNK)], idxt_v, sem))
    copies.append(pltpu.async_copy(
        len_hbm.at[pl.ds(base, LINK_CHUNK)], len_v, sem))
    for c in copies:
        c.wait()

    def body(m, carry):
        slu = pl.ds(m * 2 * LANES, LANES)
        slv = pl.ds(m * 2 * LANES + LANES, LANES)
        gu = (plsc.load_gather(arr_v, [idxh_v[slu]])
              - plsc.load_gather(arr_v, [idxt_v[slu]])) / len_v[slu]
        gv = (plsc.load_gather(arr_v, [idxh_v[slv]])
              - plsc.load_gather(arr_v, [idxt_v[slv]])) / len_v[slv]
        lo = lax.shift_right_logical(_bf16_hi(gu), 16)
        w_v[pl.ds(m * LANES, LANES)] = lo | _bf16_hi(gv)
        return carry

    lax.fori_loop(0, LINK_CHUNK // (2 * LANES), body, 0)
    pltpu.sync_copy(w_v, w_hbm.at[pl.ds(pl.multiple_of(base // 2, 32),
                                        LINK_CHUNK // 2)])


_grad_kernel = functools.partial(
    pl.kernel,
    out_type=jax.ShapeDtypeStruct((WORDS,), jnp.int32),
    mesh=_mesh,
    compiler_params=pltpu.CompilerParams(needs_layout_passes=False),
    scratch_types=[
        pltpu.VMEM((N,), jnp.float32),
        pltpu.VMEM((LINK_CHUNK,), jnp.int32),
        pltpu.VMEM((LINK_CHUNK,), jnp.int32),
        pltpu.VMEM((LINK_CHUNK,), jnp.float32),
        pltpu.VMEM((LINK_CHUNK // 2,), jnp.int32),
        pltpu.SemaphoreType.DMA,
    ],
)(_grad_body)


def _mean_body(links_hbm, w_hbm, out_hbm, w_v, idx_v, out_v, sem):
    wid = _wid()
    base = pl.multiple_of(
        jnp.where(wid == NW - 1, N - NODE_CHUNK, wid * NODE_CHUNK), 32)
    copies = [pltpu.async_copy(w_hbm.at[pl.ds(c * W_CHUNK, W_CHUNK)],
                               w_v.at[pl.ds(c * W_CHUNK, W_CHUNK)], sem)
              for c in range(ARR_SPLIT)]
    copies.append(pltpu.async_copy(
        links_hbm.at[pl.ds(pl.multiple_of(base * 4, 128), NODE_CHUNK * 4)],
        idx_v, sem))
    for c in copies:
        c.wait()

    iota4 = lax.iota(jnp.int32, LANES) * 4

    def body(i, carry):
        acc = jnp.zeros((LANES,), jnp.float32)
        for j in range(4):
            l = plsc.load_gather(idx_v, [iota4 + (i * (4 * LANES) + j)])
            k = lax.shift_left(lax.shift_right_logical(l, 5), 4) | (l & 15)
            w = plsc.load_gather(w_v, [k])
            bits = jnp.where((l & 16) != 0, w & jnp.int32(-65536),
                             lax.shift_left(w, 16))
            acc = acc + plsc.bitcast(bits, jnp.float32)
        out_v[pl.ds(i * LANES, LANES)] = acc * 0.25
        return carry

    lax.fori_loop(0, NODE_CHUNK // LANES, body, 0)
    pltpu.sync_copy(out_v, out_hbm.at[pl.ds(base, NODE_CHUNK)])


_mean_kernel = functools.partial(
    pl.kernel,
    out_type=jax.ShapeDtypeStruct((N,), jnp.float32),
    mesh=_mesh,
    compiler_params=pltpu.CompilerParams(needs_layout_passes=False),
    scratch_types=[
        pltpu.VMEM((WORDS,), jnp.int32),
        pltpu.VMEM((4 * NODE_CHUNK,), jnp.int32),
        pltpu.VMEM((NODE_CHUNK,), jnp.float32),
        pltpu.SemaphoreType.DMA,
    ],
)(_mean_body)


def kernel(array, length_of_link, node_at_link_head, node_at_link_tail,
           links_at_node):
    packed = _grad_kernel(node_at_link_head, node_at_link_tail,
                          length_of_link, array)
    return _mean_kernel(links_at_node.reshape(-1), packed)


# speedup vs baseline: 1.0023x; 1.0023x over previous
import functools

import jax
import jax.numpy as jnp
from jax import lax
from jax.experimental import pallas as pl
from jax.experimental.pallas import tpu as pltpu
from jax.experimental.pallas import tpu_sc as plsc

N = 100000
L = 200000
NW = 32
LANES = 16

LINK_CHUNK = 6272
WORDS = L // 2
NODE_CHUNK = 3136
ARR_SPLIT = 4
ARR_CHUNK = N // ARR_SPLIT
W_CHUNK = WORDS // ARR_SPLIT

_mesh = plsc.VectorSubcoreMesh(core_axis_name="c", subcore_axis_name="s")


def _wid():
    return lax.axis_index("s") * 2 + lax.axis_index("c")


def _bf16_hi(g):
    b = plsc.bitcast(g, jnp.int32)
    return (b + 0x8000) & jnp.int32(-65536)


def _grad_body(head_hbm, tail_hbm, len_hbm, array_hbm, w_hbm,
               arr_v, idxh_v, idxt_v, len_v, w_v, sem):
    wid = _wid()
    base = pl.multiple_of(
        jnp.where(wid == NW - 1, L - LINK_CHUNK, wid * LINK_CHUNK), 64)
    copies = [
        pltpu.async_copy(head_hbm.at[pl.ds(base, LINK_CHUNK)], idxh_v, sem),
        pltpu.async_copy(tail_hbm.at[pl.ds(base, LINK_CHUNK)], idxt_v, sem),
        pltpu.async_copy(len_hbm.at[pl.ds(base, LINK_CHUNK)], len_v, sem),
        pltpu.async_copy(array_hbm, arr_v, sem),
    ]
    for c in copies:
        c.wait()

    def body(m, carry):
        slu = pl.ds(m * 2 * LANES, LANES)
        slv = pl.ds(m * 2 * LANES + LANES, LANES)
        gu = (plsc.load_gather(arr_v, [idxh_v[slu]])
              - plsc.load_gather(arr_v, [idxt_v[slu]])) / len_v[slu]
        gv = (plsc.load_gather(arr_v, [idxh_v[slv]])
              - plsc.load_gather(arr_v, [idxt_v[slv]])) / len_v[slv]
        lo = lax.shift_right_logical(_bf16_hi(gu), 16)
        w_v[pl.ds(m * LANES, LANES)] = lo | _bf16_hi(gv)
        return carry

    lax.fori_loop(0, LINK_CHUNK // (2 * LANES), body, 0)
    pltpu.sync_copy(w_v, w_hbm.at[pl.ds(pl.multiple_of(base // 2, 32),
                                        LINK_CHUNK // 2)])


_grad_kernel = functools.partial(
    pl.kernel,
    out_type=jax.ShapeDtypeStruct((WORDS,), jnp.int32),
    mesh=_mesh,
    compiler_params=pltpu.CompilerParams(needs_layout_passes=False),
    scratch_types=[
        pltpu.VMEM((N,), jnp.float32),
        pltpu.VMEM((LINK_CHUNK,), jnp.int32),
        pltpu.VMEM((LINK_CHUNK,), jnp.int32),
        pltpu.VMEM((LINK_CHUNK,), jnp.float32),
        pltpu.VMEM((LINK_CHUNK // 2,), jnp.int32),
        pltpu.SemaphoreType.DMA,
    ],
)(_grad_body)


def _mean_body(links_hbm, w_hbm, out_hbm, w_v, idx_v, out_v, sem):
    wid = _wid()
    base = pl.multiple_of(
        jnp.where(wid == NW - 1, N - NODE_CHUNK, wid * NODE_CHUNK), 32)
    copies = [pltpu.async_copy(w_hbm.at[pl.ds(c * W_CHUNK, W_CHUNK)],
                               w_v.at[pl.ds(c * W_CHUNK, W_CHUNK)], sem)
              for c in range(ARR_SPLIT)]
    copies.append(pltpu.async_copy(
        links_hbm.at[pl.ds(pl.multiple_of(base * 4, 128), NODE_CHUNK * 4)],
        idx_v, sem))
    for c in copies:
        c.wait()

    iota4 = lax.iota(jnp.int32, LANES) * 4

    def body(i, carry):
        acc = jnp.zeros((LANES,), jnp.float32)
        for j in range(4):
            l = plsc.load_gather(idx_v, [iota4 + (i * (4 * LANES) + j)])
            k = lax.shift_left(lax.shift_right_logical(l, 5), 4) | (l & 15)
            w = plsc.load_gather(w_v, [k])
            bits = jnp.where((l & 16) != 0, w & jnp.int32(-65536),
                             lax.shift_left(w, 16))
            acc = acc + plsc.bitcast(bits, jnp.float32)
        out_v[pl.ds(i * LANES, LANES)] = acc * 0.25
        return carry

    lax.fori_loop(0, NODE_CHUNK // LANES, body, 0)
    pltpu.sync_copy(out_v, out_hbm.at[pl.ds(base, NODE_CHUNK)])


_mean_kernel = functools.partial(
    pl.kernel,
    out_type=jax.ShapeDtypeStruct((N,), jnp.float32),
    mesh=_mesh,
    compiler_params=pltpu.CompilerParams(needs_layout_passes=False),
    scratch_types=[
        pltpu.VMEM((WORDS,), jnp.int32),
        pltpu.VMEM((4 * NODE_CHUNK,), jnp.int32),
        pltpu.VMEM((NODE_CHUNK,), jnp.float32),
        pltpu.SemaphoreType.DMA,
    ],
)(_mean_body)


def kernel(array, length_of_link, node_at_link_head, node_at_link_tail,
           links_at_node):
    packed = _grad_kernel(node_at_link_head, node_at_link_tail,
                          length_of_link, array)
    return _mean_kernel(links_at_node.reshape(-1), packed)
